# baseline (speedup 1.0000x reference)
import functools

import jax
import jax.numpy as jnp
from jax import lax
from jax.experimental import pallas as pl
from jax.experimental.pallas import tpu as pltpu
from jax.experimental.pallas import tpu_sc as plsc

N = 10000
E = 320000
D_IN = 128
H1 = 64
H1E = H1 + 16
H2 = 32
OUT = 41

NC = 1
NS = 16
NW = NC * NS

CB = 128
E_PAD = 327680
TOT_CHUNKS = E_PAD // CB
CPS = TOT_CHUNKS // NS
N_ACC = 10240
RPS = N_ACC // NS


def _sc_agg_body(H, nbuf, *refs):
    (p_hbm, src_hbm, dst_hbm, zh_hbm, acc_out, src_v, dst_v, acc) = refs[:8]
    rows = refs[8:8 + nbuf]
    sems = refs[8 + nbuf:]

    cid = lax.axis_index("c")
    sid = lax.axis_index("s")
    base_chunk = sid * CPS
    n_chunks = CPS

    pltpu.sync_copy(zh_hbm.at[cid], acc.at[pl.ds(sid * RPS, RPS)])

    pltpu.sync_copy(src_hbm.at[pl.ds(base_chunk, CPS)], src_v)
    pltpu.sync_copy(dst_hbm.at[pl.ds(base_chunk, CPS)], dst_v)
    plsc.subcore_barrier()

    def gather_start(j, b):
        jw = jnp.where(j >= n_chunks, j - n_chunks, j)
        pltpu.async_copy(p_hbm.at[src_v.at[jw]], rows[b], sems[b])

    for b in range(nbuf):
        gather_start(jnp.int32(b), b)

    def step(g, carry):
        base = g * nbuf
        for b in range(nbuf):
            j = base + b
            pltpu.make_async_copy(p_hbm.at[src_v.at[j]], rows[b],
                                  sems[b]).wait()
            pltpu.sync_copy(rows[b], acc.at[dst_v.at[j]], add=True)
            gather_start(j + nbuf, b)
        return carry

    lax.fori_loop(0, n_chunks // nbuf, step, 0)
    for b in range(nbuf):
        pltpu.make_async_copy(p_hbm.at[src_v.at[b]], rows[b], sems[b]).wait()
    plsc.subcore_barrier()

    sl = pl.ds(sid * RPS, RPS)
    pltpu.sync_copy(acc.at[sl], acc_out.at[cid, sl])


def _make_sc_agg(H, nbuf):
    mesh = plsc.VectorSubcoreMesh(core_axis_name="c", subcore_axis_name="s",
                                  num_cores=NC, num_subcores=NS)
    scratch = [
        pltpu.VMEM((CPS, CB), jnp.int32),
        pltpu.VMEM((CPS, CB), jnp.int32),
        pltpu.VMEM_SHARED((N_ACC, H), jnp.float32),
    ]
    scratch += [pltpu.VMEM((CB, H), jnp.float32) for _ in range(nbuf)]
    scratch += [pltpu.SemaphoreType.DMA for _ in range(nbuf)]

    return pl.kernel(
        functools.partial(_sc_agg_body, H, nbuf),
        out_type=jax.ShapeDtypeStruct((NC, N_ACC, H), jnp.float32),
        mesh=mesh,
        scratch_types=tuple(scratch),
        compiler_params=pltpu.CompilerParams(use_tc_tiling_on_sc=False),
    )


def _dot(a, b):
    return lax.dot(a, b, preferred_element_type=jnp.float32)


def _pre_body(x_ref, wl_ref, wr_ref, bl_ref, p_ref, r_ref):
    xv = x_ref[...]
    p_ref[...] = jnp.concatenate(
        [_dot(xv, wl_ref[...]), jnp.ones((N, 16), jnp.float32)], axis=1)
    r_ref[...] = _dot(xv, wr_ref[...]) + bl_ref[...]


def _mean_from_partials(sp_ref, degp_ref):
    s = sp_ref[0]
    deg = degp_ref[0][:, 0:1]
    for c in range(1, NC):
        s = s + sp_ref[c]
        deg = deg + degp_ref[c][:, 0:1]
    return s * (1.0 / jnp.maximum(deg, 1.0))


def _bn_relu(z, g_ref, b_ref):
    m = jnp.mean(z, axis=0, keepdims=True)
    v = jnp.mean((z - m) ** 2, axis=0, keepdims=True)
    return jnp.maximum((z - m) * lax.rsqrt(v + 1e-5) * g_ref[...] + b_ref[...],
                       0.0)


def _mid1_body(sp_ref, degp_ref, r_ref, g_ref, b_ref, wl_ref, wr_ref, bl_ref,
               p2_ref, r2_ref):
    z = _mean_from_partials(sp_ref, degp_ref) + r_ref[...]
    h = _bn_relu(z, g_ref, b_ref)
    p2_ref[...] = _dot(h, wl_ref[...])
    r2_ref[...] = _dot(h, wr_ref[...]) + bl_ref[...]


def _mid2_body(sp_ref, degp_ref, r_ref, g_ref, b_ref, h2_ref):
    z = _mean_from_partials(sp_ref, degp_ref) + r_ref[...]
    h2_ref[...] = _bn_relu(z, g_ref, b_ref)


def _fin_body(sp_ref, degp_ref, h2_ref, wl_ref, bl_ref, wr_ref, o_ref):
    mean = _mean_from_partials(sp_ref, degp_ref)
    o = _dot(mean, wl_ref[...]) + bl_ref[...] + _dot(h2_ref[...], wr_ref[...])
    mx = jnp.max(o, axis=1, keepdims=True)
    lse = jnp.log(jnp.sum(jnp.exp(o - mx), axis=1, keepdims=True)) + mx
    o_ref[...] = o - lse


def _tc(body, out_shapes, *args):
    return pl.pallas_call(body, out_shape=out_shapes)(*args)


def kernel(x, edge_index, Wl1, bl1, Wr1, g1, b1, Wl2, bl2, Wr2, g2, b2,
           Wl3, bl3, Wr3):
    f32 = jnp.float32
    pad = E_PAD - E
    src = jnp.concatenate([edge_index[0], jnp.zeros((pad,), jnp.int32)])
    dst = jnp.concatenate([edge_index[1], jnp.full((pad,), N, jnp.int32)])
    src2d = src.reshape(E_PAD // CB, CB)
    dst2d = dst.reshape(E_PAD // CB, CB)

    z80 = jnp.zeros((NC, RPS, H1E), f32)
    z32 = jnp.zeros((NC, RPS, H2), f32)

    sc1 = _make_sc_agg(H1E, 3)
    sc2 = _make_sc_agg(H2, 8)

    p1, r1 = _tc(_pre_body,
                 (jax.ShapeDtypeStruct((N, H1E), f32),
                  jax.ShapeDtypeStruct((N, H1), f32)),
                 x, Wl1, Wr1, bl1.reshape(1, H1))

    s1e = sc1(p1, src2d, dst2d, z80)
    s1p = s1e[:, :N, :H1]
    degp = s1e[:, :N, H1:]

    p2, r2 = _tc(_mid1_body,
                 (jax.ShapeDtypeStruct((N, H2), f32),
                  jax.ShapeDtypeStruct((N, H2), f32)),
                 s1p, degp, r1, g1.reshape(1, H1), b1.reshape(1, H1),
                 Wl2, Wr2, bl2.reshape(1, H2))

    s2p = sc2(p2, src2d, dst2d, z32)[:, :N]

    h2 = _tc(_mid2_body, jax.ShapeDtypeStruct((N, H2), f32),
             s2p, degp, r2, g2.reshape(1, H2), b2.reshape(1, H2))

    s3p = sc2(h2, src2d, dst2d, z32)[:, :N]

    out = _tc(_fin_body, jax.ShapeDtypeStruct((N, OUT), f32),
              s3p, degp, h2, Wl3, bl3.reshape(1, OUT), Wr3)
    return out


# speedup vs baseline: 1.2481x; 1.2481x over previous
import functools

import jax
import jax.numpy as jnp
from jax import lax
from jax.experimental import pallas as pl
from jax.experimental.pallas import tpu as pltpu
from jax.experimental.pallas import tpu_sc as plsc

N = 10000
E = 320000
D_IN = 128
H1 = 64
H1E = H1 + 16
H2 = 32
OUT = 41

NC = 1
NS = 16
NW = NC * NS

CB = 128
E_PAD = 327680
TOT_CHUNKS = E_PAD // CB
CPS = TOT_CHUNKS // NS
N_ACC = 10240
RPS = N_ACC // NS


def _sc_agg_body(with_deg, H, nbuf, *refs):
    if with_deg:
        (p_hbm, src_hbm, dst_hbm, zh_hbm, z16_hbm, ones_hbm,
         acc_out, deg_out, src_v, dst_v, ones_v, acc, dega, dsem) = refs[:14]
        rows = refs[14:14 + nbuf]
        sems = refs[14 + nbuf:]
    else:
        (p_hbm, src_hbm, dst_hbm, zh_hbm,
         acc_out, src_v, dst_v, acc) = refs[:8]
        rows = refs[8:8 + nbuf]
        sems = refs[8 + nbuf:]

    cid = lax.axis_index("c")
    sid = lax.axis_index("s")
    base_chunk = sid * CPS
    n_chunks = CPS

    pltpu.sync_copy(zh_hbm.at[cid], acc.at[pl.ds(sid * RPS, RPS)])
    if with_deg:
        pltpu.sync_copy(z16_hbm.at[cid], dega.at[pl.ds(sid * RPS, RPS)])
        pltpu.sync_copy(ones_hbm, ones_v)

    pltpu.sync_copy(src_hbm.at[pl.ds(base_chunk, CPS)], src_v)
    pltpu.sync_copy(dst_hbm.at[pl.ds(base_chunk, CPS)], dst_v)
    plsc.subcore_barrier()

    def gather_start(j, b):
        jw = jnp.where(j >= n_chunks, j - n_chunks, j)
        pltpu.async_copy(p_hbm.at[src_v.at[jw]], rows[b], sems[b])

    for b in range(nbuf):
        gather_start(jnp.int32(b), b)

    def step(g, carry):
        base = g * nbuf
        for b in range(nbuf):
            j = base + b
            pltpu.make_async_copy(p_hbm.at[src_v.at[j]], rows[b],
                                  sems[b]).wait()
            pltpu.sync_copy(rows[b], acc.at[dst_v.at[j]], add=True)
            if with_deg:
                pltpu.async_copy(ones_v, dega.at[dst_v.at[j]], dsem)
            gather_start(j + nbuf, b)
        return carry

    lax.fori_loop(0, n_chunks // nbuf, step, 0)
    for b in range(nbuf):
        pltpu.make_async_copy(p_hbm.at[src_v.at[b]], rows[b], sems[b]).wait()
    if with_deg:
        def drain(j, carry):
            pltpu.make_async_copy(ones_v, dega.at[dst_v.at[0]], dsem).wait()
            return carry
        lax.fori_loop(0, n_chunks, drain, 0)
    plsc.subcore_barrier()

    sl = pl.ds(sid * RPS, RPS)
    pltpu.sync_copy(acc.at[sl], acc_out.at[cid, sl])
    if with_deg:
        pltpu.sync_copy(dega.at[sl], deg_out.at[cid, sl])


def _make_sc_agg(H, with_deg, nbuf):
    mesh = plsc.VectorSubcoreMesh(core_axis_name="c", subcore_axis_name="s",
                                  num_cores=NC, num_subcores=NS)
    out_type = [jax.ShapeDtypeStruct((NC, N_ACC, H), jnp.float32)]
    scratch = [
        pltpu.VMEM((CPS, CB), jnp.int32),
        pltpu.VMEM((CPS, CB), jnp.int32),
    ]
    if with_deg:
        out_type.append(jax.ShapeDtypeStruct((NC, N_ACC, 16), jnp.float32))
        scratch += [
            pltpu.VMEM((CB, 16), jnp.float32),
            pltpu.VMEM_SHARED((N_ACC, H), jnp.float32),
            pltpu.VMEM_SHARED((N_ACC, 16), jnp.float32),
            pltpu.SemaphoreType.DMA,
        ]
    else:
        scratch.append(pltpu.VMEM_SHARED((N_ACC, H), jnp.float32))
    scratch += [pltpu.VMEM((CB, H), jnp.float32) for _ in range(nbuf)]
    scratch += [pltpu.SemaphoreType.DMA for _ in range(nbuf)]

    return pl.kernel(
        functools.partial(_sc_agg_body, with_deg, H, nbuf),
        out_type=tuple(out_type) if with_deg else out_type[0],
        mesh=mesh,
        scratch_types=tuple(scratch),
        compiler_params=pltpu.CompilerParams(use_tc_tiling_on_sc=False),
    )


def _dot(a, b):
    return lax.dot(a, b, preferred_element_type=jnp.float32)


def _pre_body(x_ref, wl_ref, wr_ref, bl_ref, p_ref, r_ref):
    xv = x_ref[...]
    p_ref[...] = _dot(xv, wl_ref[...])
    r_ref[...] = _dot(xv, wr_ref[...]) + bl_ref[...]


def _mean_from_partials(sp_ref, degp_ref):
    s = sp_ref[0]
    deg = degp_ref[0][:, 0:1]
    for c in range(1, NC):
        s = s + sp_ref[c]
        deg = deg + degp_ref[c][:, 0:1]
    return s * (1.0 / jnp.maximum(deg, 1.0))


def _bn_relu(z, g_ref, b_ref):
    m = jnp.mean(z, axis=0, keepdims=True)
    v = jnp.mean((z - m) ** 2, axis=0, keepdims=True)
    return jnp.maximum((z - m) * lax.rsqrt(v + 1e-5) * g_ref[...] + b_ref[...],
                       0.0)


def _mid1_body(sp_ref, degp_ref, r_ref, g_ref, b_ref, wl_ref, wr_ref, bl_ref,
               p2_ref, r2_ref):
    z = _mean_from_partials(sp_ref, degp_ref) + r_ref[...]
    h = _bn_relu(z, g_ref, b_ref)
    p2_ref[...] = _dot(h, wl_ref[...])
    r2_ref[...] = _dot(h, wr_ref[...]) + bl_ref[...]


def _mid2_body(sp_ref, degp_ref, r_ref, g_ref, b_ref, h2_ref):
    z = _mean_from_partials(sp_ref, degp_ref) + r_ref[...]
    h2_ref[...] = _bn_relu(z, g_ref, b_ref)


def _fin_body(sp_ref, degp_ref, h2_ref, wl_ref, bl_ref, wr_ref, o_ref):
    mean = _mean_from_partials(sp_ref, degp_ref)
    o = _dot(mean, wl_ref[...]) + bl_ref[...] + _dot(h2_ref[...], wr_ref[...])
    mx = jnp.max(o, axis=1, keepdims=True)
    lse = jnp.log(jnp.sum(jnp.exp(o - mx), axis=1, keepdims=True)) + mx
    o_ref[...] = o - lse


def _tc(body, out_shapes, *args):
    return pl.pallas_call(body, out_shape=out_shapes)(*args)


def kernel(x, edge_index, Wl1, bl1, Wr1, g1, b1, Wl2, bl2, Wr2, g2, b2,
           Wl3, bl3, Wr3):
    f32 = jnp.float32
    pad = E_PAD - E
    src = jnp.concatenate([edge_index[0], jnp.zeros((pad,), jnp.int32)])
    dst = jnp.concatenate([edge_index[1], jnp.full((pad,), N, jnp.int32)])
    src2d = src.reshape(E_PAD // CB, CB)
    dst2d = dst.reshape(E_PAD // CB, CB)

    z64 = jnp.zeros((NC, RPS, H1), f32)
    z32 = jnp.zeros((NC, RPS, H2), f32)
    z16 = jnp.zeros((NC, RPS, 16), f32)
    ones16 = jnp.ones((CB, 16), f32)

    sc1 = _make_sc_agg(H1, True, 4)
    sc2 = _make_sc_agg(H2, False, 8)

    p1, r1 = _tc(_pre_body,
                 (jax.ShapeDtypeStruct((N, H1), f32),
                  jax.ShapeDtypeStruct((N, H1), f32)),
                 x, Wl1, Wr1, bl1.reshape(1, H1))

    s1p, degp = sc1(p1, src2d, dst2d, z64, z16, ones16)
    s1p = s1p[:, :N]
    degp = degp[:, :N]

    p2, r2 = _tc(_mid1_body,
                 (jax.ShapeDtypeStruct((N, H2), f32),
                  jax.ShapeDtypeStruct((N, H2), f32)),
                 s1p, degp, r1, g1.reshape(1, H1), b1.reshape(1, H1),
                 Wl2, Wr2, bl2.reshape(1, H2))

    s2p = sc2(p2, src2d, dst2d, z32)[:, :N]

    h2 = _tc(_mid2_body, jax.ShapeDtypeStruct((N, H2), f32),
             s2p, degp, r2, g2.reshape(1, H2), b2.reshape(1, H2))

    s3p = sc2(h2, src2d, dst2d, z32)[:, :N]

    out = _tc(_fin_body, jax.ShapeDtypeStruct((N, OUT), f32),
              s3p, degp, h2, Wl3, bl3.reshape(1, OUT), Wr3)
    return out
